# baseline (speedup 1.0000x reference)
import functools

import jax
import jax.numpy as jnp
from jax import lax
from jax.experimental import pallas as pl
from jax.experimental.pallas import tpu as pltpu
from jax.experimental.pallas import tpu_sc as plsc

VOCAB = 30522
HIDDEN = 768
MAX_POS = 512
BATCH = 32
SEQ = 512
EPS = 1e-12

NC = 2
NS = 16
NW = NC * NS
H = 2
BCH = BATCH // H
TOKENS_CH = BCH * SEQ
TPW = TOKENS_CH // NW
CHUNK = 64
NCHUNK = TPW // CHUNK


def _make_gather(tpw):
    nchunk = tpw // CHUNK

    def _gather_body(table_hbm, idx_hbm, out_hbm, idx_v, buf0, buf1, sem0, sem1):
        wid = lax.axis_index("s") * NC + lax.axis_index("c")
        pltpu.sync_copy(idx_hbm.at[wid], idx_v)
        bufs = (buf0, buf1)
        sems = (sem0, sem1)
        copies = [None, None]
        copies[0] = pltpu.async_copy(table_hbm.at[idx_v.at[0]], bufs[0], sems[0])
        base = wid * tpw
        for c in range(nchunk):
            if c + 1 < nchunk:
                copies[(c + 1) % 2] = pltpu.async_copy(
                    table_hbm.at[idx_v.at[c + 1]], bufs[(c + 1) % 2],
                    sems[(c + 1) % 2])
            copies[c % 2].wait()
            pltpu.sync_copy(bufs[c % 2], out_hbm.at[pl.ds(base + c * CHUNK, CHUNK)])

    return functools.partial(
        pl.kernel,
        mesh=plsc.VectorSubcoreMesh(core_axis_name="c", subcore_axis_name="s"),
        out_type=jax.ShapeDtypeStruct((NW * tpw, HIDDEN), jnp.float32),
        scratch_types=[
            pltpu.VMEM((nchunk, CHUNK), jnp.int32),
            pltpu.VMEM((CHUNK, HIDDEN), jnp.float32),
            pltpu.VMEM((CHUNK, HIDDEN), jnp.float32),
            pltpu.SemaphoreType.DMA,
            pltpu.SemaphoreType.DMA,
        ],
    )(_gather_body)


_sc_gather = _make_gather(TPW)


ROWS = 2


def _ln_body(words_ref, tt_ref, pos_ref, type_ref, gamma_ref, beta_ref, *rest):
    out_ref = rest[-1]
    for r in range(ROWS):
        x = words_ref[r]
        tt = tt_ref[r, 0].astype(jnp.float32)
        t0 = type_ref[0]
        t1 = type_ref[1]
        ttb = lax.broadcast_in_dim(tt, (SEQ, HIDDEN), (0,))
        tsel = t0[None, :] + ttb * (t1 - t0)[None, :]
        x = x + pos_ref[...] + tsel
        mean = jnp.mean(x, axis=-1, keepdims=True)
        xc = x - mean
        var = jnp.mean(xc * xc, axis=-1, keepdims=True)
        inv = lax.rsqrt(var + EPS)
        out_ref[r] = (xc * inv) * gamma_ref[...] + beta_ref[...]


def _make_ln(h):
    aliased = h > 0
    in_specs = [
        pl.BlockSpec((ROWS, SEQ, HIDDEN), lambda b: (b, 0, 0)),
        pl.BlockSpec((ROWS, 1, SEQ), lambda b: (b, 0, 0)),
        pl.BlockSpec((SEQ, HIDDEN), lambda b: (0, 0)),
        pl.BlockSpec((2, HIDDEN), lambda b: (0, 0)),
        pl.BlockSpec((1, HIDDEN), lambda b: (0, 0)),
        pl.BlockSpec((1, HIDDEN), lambda b: (0, 0)),
    ]
    if aliased:
        in_specs.append(pl.BlockSpec(memory_space=pl.ANY))
    return pl.pallas_call(
        _ln_body,
        grid=(BCH // ROWS,),
        in_specs=in_specs,
        out_specs=pl.BlockSpec((ROWS, SEQ, HIDDEN),
                               lambda b, _h=h: (b + _h * (BCH // ROWS), 0, 0)),
        out_shape=jax.ShapeDtypeStruct((BATCH, SEQ, HIDDEN), jnp.float32),
        input_output_aliases={6: 0} if aliased else {},
    )


_ln_calls = [_make_ln(h) for h in range(H)]


def kernel(input_ids, token_type_ids, W_word, W_pos, W_type, gamma, beta):
    idx = input_ids.reshape(H, NW, NCHUNK, CHUNK).astype(jnp.int32)
    tt = token_type_ids.reshape(H, BCH, 1, SEQ).astype(jnp.int32)
    gamma2 = gamma.reshape(1, HIDDEN)
    beta2 = beta.reshape(1, HIDDEN)
    words = [_sc_gather(W_word, idx[h]).reshape(BCH, SEQ, HIDDEN)
             for h in range(H)]
    out = None
    for h in range(H):
        args = (words[h], tt[h], W_pos, W_type, gamma2, beta2)
        out = _ln_calls[h](*args) if out is None else _ln_calls[h](*args, out)
    return out


# speedup vs baseline: 1.0175x; 1.0175x over previous
import functools

import jax
import jax.numpy as jnp
from jax import lax
from jax.experimental import pallas as pl
from jax.experimental.pallas import tpu as pltpu
from jax.experimental.pallas import tpu_sc as plsc

VOCAB = 30522
HIDDEN = 768
MAX_POS = 512
BATCH = 32
SEQ = 512
EPS = 1e-12

NC = 2
NS = 16
NW = NC * NS
H = 2
BCH = BATCH // H
TOKENS_CH = BCH * SEQ
TPW = TOKENS_CH // NW
CHUNK = 64
NCHUNK = TPW // CHUNK


def _make_gather(tpw):
    nchunk = tpw // CHUNK

    def _gather_body(table_hbm, idx_hbm, out_hbm, idx_v, buf0, buf1, sem0, sem1):
        wid = lax.axis_index("s") * NC + lax.axis_index("c")
        pltpu.sync_copy(idx_hbm.at[wid], idx_v)
        bufs = (buf0, buf1)
        sems = (sem0, sem1)
        copies = [None, None]
        copies[0] = pltpu.async_copy(table_hbm.at[idx_v.at[0]], bufs[0], sems[0])
        base = wid * tpw
        for c in range(nchunk):
            if c + 1 < nchunk:
                copies[(c + 1) % 2] = pltpu.async_copy(
                    table_hbm.at[idx_v.at[c + 1]], bufs[(c + 1) % 2],
                    sems[(c + 1) % 2])
            copies[c % 2].wait()
            pltpu.sync_copy(bufs[c % 2], out_hbm.at[pl.ds(base + c * CHUNK, CHUNK)])

    return functools.partial(
        pl.kernel,
        mesh=plsc.VectorSubcoreMesh(core_axis_name="c", subcore_axis_name="s"),
        out_type=jax.ShapeDtypeStruct((NW * tpw, HIDDEN), jnp.float32),
        scratch_types=[
            pltpu.VMEM((nchunk, CHUNK), jnp.int32),
            pltpu.VMEM((CHUNK, HIDDEN), jnp.float32),
            pltpu.VMEM((CHUNK, HIDDEN), jnp.float32),
            pltpu.SemaphoreType.DMA,
            pltpu.SemaphoreType.DMA,
        ],
    )(_gather_body)


_sc_gather = _make_gather(TPW)


ROWS = 4


def _ln_body(words_ref, tt_ref, pos_ref, type_ref, gamma_ref, beta_ref, *rest):
    out_ref = rest[-1]
    for r in range(ROWS):
        x = words_ref[r]
        tt = tt_ref[r, 0].astype(jnp.float32)
        t0 = type_ref[0]
        t1 = type_ref[1]
        ttb = lax.broadcast_in_dim(tt, (SEQ, HIDDEN), (0,))
        tsel = t0[None, :] + ttb * (t1 - t0)[None, :]
        x = x + pos_ref[...] + tsel
        mean = jnp.mean(x, axis=-1, keepdims=True)
        xc = x - mean
        var = jnp.mean(xc * xc, axis=-1, keepdims=True)
        inv = lax.rsqrt(var + EPS)
        out_ref[r] = (xc * inv) * gamma_ref[...] + beta_ref[...]


def _make_ln(h):
    aliased = h > 0
    in_specs = [
        pl.BlockSpec((ROWS, SEQ, HIDDEN), lambda b: (b, 0, 0)),
        pl.BlockSpec((ROWS, 1, SEQ), lambda b: (b, 0, 0)),
        pl.BlockSpec((SEQ, HIDDEN), lambda b: (0, 0)),
        pl.BlockSpec((2, HIDDEN), lambda b: (0, 0)),
        pl.BlockSpec((1, HIDDEN), lambda b: (0, 0)),
        pl.BlockSpec((1, HIDDEN), lambda b: (0, 0)),
    ]
    if aliased:
        in_specs.append(pl.BlockSpec(memory_space=pl.ANY))
    return pl.pallas_call(
        _ln_body,
        grid=(BCH // ROWS,),
        in_specs=in_specs,
        out_specs=pl.BlockSpec((ROWS, SEQ, HIDDEN),
                               lambda b, _h=h: (b + _h * (BCH // ROWS), 0, 0)),
        out_shape=jax.ShapeDtypeStruct((BATCH, SEQ, HIDDEN), jnp.float32),
        input_output_aliases={6: 0} if aliased else {},
    )


_ln_calls = [_make_ln(h) for h in range(H)]


def kernel(input_ids, token_type_ids, W_word, W_pos, W_type, gamma, beta):
    idx = input_ids.reshape(H, NW, NCHUNK, CHUNK).astype(jnp.int32)
    tt = token_type_ids.reshape(H, BCH, 1, SEQ).astype(jnp.int32)
    gamma2 = gamma.reshape(1, HIDDEN)
    beta2 = beta.reshape(1, HIDDEN)
    words = [_sc_gather(W_word, idx[h]).reshape(BCH, SEQ, HIDDEN)
             for h in range(H)]
    out = None
    for h in range(H):
        args = (words[h], tt[h], W_pos, W_type, gamma2, beta2)
        out = _ln_calls[h](*args) if out is None else _ln_calls[h](*args, out)
    return out
